# baseline (speedup 1.0000x reference)
import functools

import jax
import jax.numpy as jnp
from jax import lax
from jax.experimental import pallas as pl
from jax.experimental.pallas import tpu as pltpu
from jax.experimental.pallas import tpu_sc as plsc

N = 10000
E = 320000
D = 128
D2 = 2 * D

NC = 2
NS = 16
NW = NC * NS
_SPLITS = ((0, 128000), (128000, 192000))
CH = 40
_RING = 4
NPAD = 10240
RPT = NPAD // NS


@functools.lru_cache(maxsize=None)
def _mesh():
    return plsc.VectorSubcoreMesh(
        core_axis_name="c", subcore_axis_name="s",
        num_cores=NC, num_subcores=NS)


def _sigmoid(v):
    return 1.0 / (1.0 + jnp.exp(-v))


def _softplus(v):
    return jnp.maximum(v, 0.0) + jnp.log(1.0 + jnp.exp(-jnp.abs(v)))


def _pack2(v):
    lo = lax.bitcast_convert_type(v[:, :D].astype(jnp.bfloat16), jnp.uint16)
    hi = lax.bitcast_convert_type(v[:, D:].astype(jnp.bfloat16), jnp.uint16)
    w = lo.astype(jnp.uint32) | (hi.astype(jnp.uint32) << 16)
    return lax.bitcast_convert_type(w, jnp.int32)


_BN = 2000


def _pq_body(x_ref, ws_ref, wd_ref, b_ref, p_ref, q_ref):
    xb = x_ref[...]
    p = jnp.dot(xb, ws_ref[...],
                preferred_element_type=jnp.float32) + b_ref[...]
    q = jnp.dot(xb, wd_ref[...], preferred_element_type=jnp.float32)
    p_ref[...] = _pack2(p)
    q_ref[...] = _pack2(q)


def _pq_call(x, wsrc, wdst, b2):
    return pl.pallas_call(
        _pq_body,
        grid=(N // _BN,),
        in_specs=[
            pl.BlockSpec((_BN, D), lambda i: (i, 0)),
            pl.BlockSpec((D, D2), lambda i: (0, 0)),
            pl.BlockSpec((D, D2), lambda i: (0, 0)),
            pl.BlockSpec((1, D2), lambda i: (0, 0)),
        ],
        out_specs=[
            pl.BlockSpec((_BN, D), lambda i: (i, 0)),
            pl.BlockSpec((_BN, D), lambda i: (i, 0)),
        ],
        out_shape=[
            jax.ShapeDtypeStruct((N, D), jnp.int32),
            jax.ShapeDtypeStruct((N, D), jnp.int32),
        ],
    )(x, wsrc, wdst, b2)


def _ring_pipeline(nchunk, issue, wait_in, consume, wait_out):
    quads = nchunk // _RING
    tail = nchunk % _RING

    for b in range(_RING - 1):
        issue(b, b)

    def quad(i4, carry):
        for b in range(_RING):
            c = i4 * _RING + b
            tb = (b + _RING - 1) % _RING

            @pl.when(c >= 1)
            def _():
                wait_out(tb)

            @pl.when(c + (_RING - 1) < nchunk)
            def _():
                issue(c + (_RING - 1), tb)

            wait_in(b)
            consume(c, b)
        return carry

    lax.fori_loop(0, quads, quad, 0)

    for c in range(quads * _RING, nchunk):
        b = c % _RING
        wait_out((b + _RING - 1) % _RING)
        if c + (_RING - 1) < nchunk:
            issue(c + (_RING - 1), (b + _RING - 1) % _RING)
        wait_in(b)
        consume(c, b)

    wait_out((nchunk - 1) % _RING)


def _make_gather_body(e0, epw):
    nchunk = epw // CH

    def body(p_hbm, q_hbm, src_hbm, dst_hbm, gs_hbm, gd_hbm, *bufs):
        sidx, didx = bufs[0], bufs[1]
        pbs = bufs[2:2 + _RING]
        qbs = bufs[2 + _RING:2 + 2 * _RING]
        gsems = bufs[2 + 2 * _RING:2 + 3 * _RING]
        wsems = bufs[2 + 3 * _RING:2 + 4 * _RING]

        wid = lax.axis_index("s") * NC + lax.axis_index("c")
        lbase = wid * epw
        pltpu.sync_copy(src_hbm.at[pl.ds(e0 + lbase, epw)], sidx)
        pltpu.sync_copy(dst_hbm.at[pl.ds(e0 + lbase, epw)], didx)

        def issue(c, b):
            isl = pl.ds(c * CH, CH)
            pltpu.async_copy(p_hbm.at[sidx.at[isl]], pbs[b], gsems[b])
            pltpu.async_copy(q_hbm.at[didx.at[isl]], qbs[b], gsems[b])

        def wait_in(b):
            pltpu.make_async_copy(p_hbm.at[sidx.at[pl.ds(0, CH)]], pbs[b],
                                  gsems[b]).wait()
            pltpu.make_async_copy(q_hbm.at[didx.at[pl.ds(0, CH)]], qbs[b],
                                  gsems[b]).wait()

        def consume(c, b):
            osl = pl.ds(lbase + c * CH, CH)
            pltpu.async_copy(pbs[b], gs_hbm.at[osl], wsems[b])
            pltpu.async_copy(qbs[b], gd_hbm.at[osl], wsems[b])

        def wait_out(b):
            pltpu.make_async_copy(pbs[b], gs_hbm.at[pl.ds(0, CH)],
                                  wsems[b]).wait()
            pltpu.make_async_copy(qbs[b], gd_hbm.at[pl.ds(0, CH)],
                                  wsems[b]).wait()

        _ring_pipeline(nchunk, issue, wait_in, consume, wait_out)

    return body


@functools.lru_cache(maxsize=None)
def _gather_kernel(h):
    e0, esz = _SPLITS[h]
    return pl.kernel(
        _make_gather_body(e0, esz // NW),
        out_type=[
            jax.ShapeDtypeStruct((esz, D), jnp.int32),
            jax.ShapeDtypeStruct((esz, D), jnp.int32),
        ],
        mesh=_mesh(),
        scratch_types=(
            [pltpu.VMEM((esz // NW,), jnp.int32)] * 2
            + [pltpu.VMEM((CH, D), jnp.int32)] * (2 * _RING)
            + [pltpu.SemaphoreType.DMA] * (2 * _RING)
        ),
    )


_BE = 1280


def _edge_body(ea_ref, gs_ref, gd_ref, we_ref, m_ref):
    ll = jnp.dot(ea_ref[...], we_ref[...], preferred_element_type=jnp.float32)
    gs = gs_ref[...]
    gd = gd_ref[...]
    lf = (lax.bitcast_convert_type(gs << 16, jnp.float32)
          + lax.bitcast_convert_type(gd << 16, jnp.float32))
    ls = (lax.bitcast_convert_type(gs & jnp.int32(-65536), jnp.float32)
          + lax.bitcast_convert_type(gd & jnp.int32(-65536), jnp.float32))
    f = _sigmoid(ll[:, :D] + lf)
    s = _softplus(ll[:, D:] + ls)
    m_ref[...] = f * s


def _edge_call(ea, gs, gd, wea, h):
    e0, esz = _SPLITS[h]
    hoff = e0 // _BE
    return pl.pallas_call(
        _edge_body,
        grid=(esz // _BE,),
        in_specs=[
            pl.BlockSpec((_BE, D), lambda i: (i + hoff, 0)),
            pl.BlockSpec((_BE, D), lambda i: (i, 0)),
            pl.BlockSpec((_BE, D), lambda i: (i, 0)),
            pl.BlockSpec((D, D2), lambda i: (0, 0)),
        ],
        out_specs=pl.BlockSpec((_BE, D), lambda i: (i, 0)),
        out_shape=jax.ShapeDtypeStruct((esz, D), jnp.float32),
    )(ea, gs, gd, wea)


def _make_scatter_body(e0, epw):
    nchunk = epw // CH

    def body(m_hbm, src_hbm, out_hbm, *bufs):
        idxbs = bufs[:_RING]
        mbufs = bufs[_RING:2 * _RING]
        acc_sh = bufs[2 * _RING]
        lsems = bufs[2 * _RING + 1:3 * _RING + 1]
        ssems = bufs[3 * _RING + 1:4 * _RING + 1]

        cid = lax.axis_index("c")
        sid = lax.axis_index("s")
        wid = sid * NC + cid
        lbase = wid * epw

        def zrow(r, c):
            for j in range(D // 16):
                mbufs[0][r, pl.ds(j * 16, 16)] = jnp.zeros((16,), jnp.float32)
            return c

        lax.fori_loop(0, CH, zrow, 0)
        for t in range(RPT // CH):
            pltpu.sync_copy(mbufs[0],
                            acc_sh.at[pl.ds(sid * RPT + t * CH, CH)])
        plsc.subcore_barrier()

        def issue(c, b):
            pltpu.async_copy(m_hbm.at[pl.ds(lbase + c * CH, CH)],
                             mbufs[b], lsems[b])
            pltpu.async_copy(src_hbm.at[pl.ds(e0 + lbase + c * CH, CH)],
                             idxbs[b], lsems[b])

        def wait_in(b):
            pltpu.make_async_copy(m_hbm.at[pl.ds(0, CH)], mbufs[b],
                                  lsems[b]).wait()
            pltpu.make_async_copy(src_hbm.at[pl.ds(0, CH)], idxbs[b],
                                  lsems[b]).wait()

        def consume(c, b):
            pltpu.async_copy(mbufs[b], acc_sh.at[idxbs[b]], ssems[b],
                             add=True)

        def wait_out(b):
            pltpu.make_async_copy(mbufs[b], acc_sh.at[idxbs[b]],
                                  ssems[b]).wait()

        _ring_pipeline(nchunk, issue, wait_in, consume, wait_out)
        plsc.subcore_barrier()

        for t in range(RPT // CH):
            row0 = sid * RPT + t * CH
            pltpu.sync_copy(acc_sh.at[pl.ds(row0, CH)],
                            out_hbm.at[pl.ds(cid * NPAD + row0, CH)])

    return body


@functools.lru_cache(maxsize=None)
def _scatter_kernel(h):
    e0, esz = _SPLITS[h]
    return pl.kernel(
        _make_scatter_body(e0, esz // NW),
        out_type=jax.ShapeDtypeStruct((NC * NPAD, D), jnp.float32),
        mesh=_mesh(),
        scratch_types=(
            [pltpu.VMEM((CH,), jnp.int32)] * _RING
            + [pltpu.VMEM((CH, D), jnp.float32)] * _RING
            + [pltpu.VMEM_SHARED((NPAD, D), jnp.float32)]
            + [pltpu.SemaphoreType.DMA] * (2 * _RING)
        ),
    )


def _final_body(*args):
    mp_refs = args[:len(_SPLITS)]
    x_ref, gam_ref, bet_ref, o_ref = args[len(_SPLITS):]
    msg = mp_refs[0][0:N, :] + mp_refs[0][NPAD:NPAD + N, :]
    for mp in mp_refs[1:]:
        msg = msg + mp[0:N, :] + mp[NPAD:NPAD + N, :]
    mean = jnp.mean(msg, axis=0, keepdims=True)
    var = jnp.mean((msg - mean) ** 2, axis=0, keepdims=True)
    norm = (msg - mean) / jnp.sqrt(var + 1e-5) * gam_ref[...] + bet_ref[...]
    o_ref[...] = _softplus(x_ref[...] + norm)


def _final_call(mps, x, gamma, beta):
    return pl.pallas_call(
        _final_body,
        out_shape=jax.ShapeDtypeStruct((N, D), jnp.float32),
    )(*mps, x, gamma, beta)


def kernel(x, edge_attr, edge_source, edge_target, Wf, bf, Ws, bs, gamma, beta):
    src = edge_source.astype(jnp.int32)
    dst = edge_target.astype(jnp.int32)
    b2 = jnp.concatenate([bf, bs]).reshape(1, D2)
    wsrc = jnp.concatenate([Wf[:, :D].T, Ws[:, :D].T], axis=1)
    wdst = jnp.concatenate([Wf[:, D:2 * D].T, Ws[:, D:2 * D].T], axis=1)
    wea = jnp.concatenate([Wf[:, 2 * D:].T, Ws[:, 2 * D:].T], axis=1)

    p, q = _pq_call(x, wsrc, wdst, b2)
    mps = []
    gg = [_gather_kernel(h)(p, q, src, dst) for h in range(len(_SPLITS))]
    for h, (gs, gd) in enumerate(gg):
        m = _edge_call(edge_attr, gs, gd, wea, h)
        mps.append(_scatter_kernel(h)(m, src))
    return _final_call(mps, x, gamma.reshape(1, D), beta.reshape(1, D))


# speedup vs baseline: 1.0030x; 1.0030x over previous
import functools

import jax
import jax.numpy as jnp
from jax import lax
from jax.experimental import pallas as pl
from jax.experimental.pallas import tpu as pltpu
from jax.experimental.pallas import tpu_sc as plsc

N = 10000
E = 320000
D = 128
D2 = 2 * D

NC = 2
NS = 16
NW = NC * NS
_SPLITS = ((0, 168960), (168960, 151040))
CH = 80
_RING = 4
NPAD = 10240
RPT = NPAD // NS


@functools.lru_cache(maxsize=None)
def _mesh():
    return plsc.VectorSubcoreMesh(
        core_axis_name="c", subcore_axis_name="s",
        num_cores=NC, num_subcores=NS)


def _sigmoid(v):
    return 1.0 / (1.0 + jnp.exp(-v))


def _softplus(v):
    return jnp.maximum(v, 0.0) + jnp.log(1.0 + jnp.exp(-jnp.abs(v)))


def _pack2(v):
    lo = lax.bitcast_convert_type(v[:, :D].astype(jnp.bfloat16), jnp.uint16)
    hi = lax.bitcast_convert_type(v[:, D:].astype(jnp.bfloat16), jnp.uint16)
    w = lo.astype(jnp.uint32) | (hi.astype(jnp.uint32) << 16)
    return lax.bitcast_convert_type(w, jnp.int32)


_BN = 2000


def _pq_body(x_ref, ws_ref, wd_ref, b_ref, p_ref, q_ref):
    xb = x_ref[...]
    p = jnp.dot(xb, ws_ref[...],
                preferred_element_type=jnp.float32) + b_ref[...]
    q = jnp.dot(xb, wd_ref[...], preferred_element_type=jnp.float32)
    p_ref[...] = _pack2(p)
    q_ref[...] = _pack2(q)


def _pq_call(x, wsrc, wdst, b2):
    return pl.pallas_call(
        _pq_body,
        grid=(N // _BN,),
        in_specs=[
            pl.BlockSpec((_BN, D), lambda i: (i, 0)),
            pl.BlockSpec((D, D2), lambda i: (0, 0)),
            pl.BlockSpec((D, D2), lambda i: (0, 0)),
            pl.BlockSpec((1, D2), lambda i: (0, 0)),
        ],
        out_specs=[
            pl.BlockSpec((_BN, D), lambda i: (i, 0)),
            pl.BlockSpec((_BN, D), lambda i: (i, 0)),
        ],
        out_shape=[
            jax.ShapeDtypeStruct((N, D), jnp.int32),
            jax.ShapeDtypeStruct((N, D), jnp.int32),
        ],
    )(x, wsrc, wdst, b2)


def _ring_pipeline(nchunk, issue, wait_in, consume, wait_out):
    quads = nchunk // _RING
    tail = nchunk % _RING

    for b in range(_RING - 1):
        issue(b, b)

    def quad(i4, carry):
        for b in range(_RING):
            c = i4 * _RING + b
            tb = (b + _RING - 1) % _RING

            @pl.when(c >= 1)
            def _():
                wait_out(tb)

            @pl.when(c + (_RING - 1) < nchunk)
            def _():
                issue(c + (_RING - 1), tb)

            wait_in(b)
            consume(c, b)
        return carry

    lax.fori_loop(0, quads, quad, 0)

    for c in range(quads * _RING, nchunk):
        b = c % _RING
        wait_out((b + _RING - 1) % _RING)
        if c + (_RING - 1) < nchunk:
            issue(c + (_RING - 1), (b + _RING - 1) % _RING)
        wait_in(b)
        consume(c, b)

    wait_out((nchunk - 1) % _RING)


def _make_gather_body(e0, epw):
    nchunk = epw // CH

    def body(p_hbm, q_hbm, src_hbm, dst_hbm, gs_hbm, gd_hbm, *bufs):
        sidx, didx = bufs[0], bufs[1]
        pbs = bufs[2:2 + _RING]
        qbs = bufs[2 + _RING:2 + 2 * _RING]
        gsems = bufs[2 + 2 * _RING:2 + 3 * _RING]
        wsems = bufs[2 + 3 * _RING:2 + 4 * _RING]

        wid = lax.axis_index("s") * NC + lax.axis_index("c")
        lbase = wid * epw
        pltpu.sync_copy(src_hbm.at[pl.ds(e0 + lbase, epw)], sidx)
        pltpu.sync_copy(dst_hbm.at[pl.ds(e0 + lbase, epw)], didx)

        def issue(c, b):
            isl = pl.ds(c * CH, CH)
            pltpu.async_copy(p_hbm.at[sidx.at[isl]], pbs[b], gsems[b])
            pltpu.async_copy(q_hbm.at[didx.at[isl]], qbs[b], gsems[b])

        def wait_in(b):
            pltpu.make_async_copy(p_hbm.at[sidx.at[pl.ds(0, CH)]], pbs[b],
                                  gsems[b]).wait()
            pltpu.make_async_copy(q_hbm.at[didx.at[pl.ds(0, CH)]], qbs[b],
                                  gsems[b]).wait()

        def consume(c, b):
            osl = pl.ds(lbase + c * CH, CH)
            pltpu.async_copy(pbs[b], gs_hbm.at[osl], wsems[b])
            pltpu.async_copy(qbs[b], gd_hbm.at[osl], wsems[b])

        def wait_out(b):
            pltpu.make_async_copy(pbs[b], gs_hbm.at[pl.ds(0, CH)],
                                  wsems[b]).wait()
            pltpu.make_async_copy(qbs[b], gd_hbm.at[pl.ds(0, CH)],
                                  wsems[b]).wait()

        _ring_pipeline(nchunk, issue, wait_in, consume, wait_out)

    return body


@functools.lru_cache(maxsize=None)
def _gather_kernel(h):
    e0, esz = _SPLITS[h]
    return pl.kernel(
        _make_gather_body(e0, esz // NW),
        out_type=[
            jax.ShapeDtypeStruct((esz, D), jnp.int32),
            jax.ShapeDtypeStruct((esz, D), jnp.int32),
        ],
        mesh=_mesh(),
        scratch_types=(
            [pltpu.VMEM((esz // NW,), jnp.int32)] * 2
            + [pltpu.VMEM((CH, D), jnp.int32)] * (2 * _RING)
            + [pltpu.SemaphoreType.DMA] * (2 * _RING)
        ),
    )


_BE = 1280


def _edge_body(ea_ref, gs_ref, gd_ref, we_ref, m_ref):
    ll = jnp.dot(ea_ref[...], we_ref[...], preferred_element_type=jnp.float32)
    gs = gs_ref[...]
    gd = gd_ref[...]
    lf = (lax.bitcast_convert_type(gs << 16, jnp.float32)
          + lax.bitcast_convert_type(gd << 16, jnp.float32))
    ls = (lax.bitcast_convert_type(gs & jnp.int32(-65536), jnp.float32)
          + lax.bitcast_convert_type(gd & jnp.int32(-65536), jnp.float32))
    f = _sigmoid(ll[:, :D] + lf)
    s = _softplus(ll[:, D:] + ls)
    m_ref[...] = f * s


def _edge_call(ea, gs, gd, wea, h):
    e0, esz = _SPLITS[h]
    hoff = e0 // _BE
    return pl.pallas_call(
        _edge_body,
        grid=(esz // _BE,),
        in_specs=[
            pl.BlockSpec((_BE, D), lambda i: (i + hoff, 0)),
            pl.BlockSpec((_BE, D), lambda i: (i, 0)),
            pl.BlockSpec((_BE, D), lambda i: (i, 0)),
            pl.BlockSpec((D, D2), lambda i: (0, 0)),
        ],
        out_specs=pl.BlockSpec((_BE, D), lambda i: (i, 0)),
        out_shape=jax.ShapeDtypeStruct((esz, D), jnp.float32),
    )(ea, gs, gd, wea)


def _make_scatter_body(e0, epw):
    nchunk = epw // CH

    def body(m_hbm, src_hbm, out_hbm, *bufs):
        idxbs = bufs[:_RING]
        mbufs = bufs[_RING:2 * _RING]
        acc_sh = bufs[2 * _RING]
        lsems = bufs[2 * _RING + 1:3 * _RING + 1]
        ssems = bufs[3 * _RING + 1:4 * _RING + 1]

        cid = lax.axis_index("c")
        sid = lax.axis_index("s")
        wid = sid * NC + cid
        lbase = wid * epw

        def zrow(r, c):
            for j in range(D // 16):
                mbufs[0][r, pl.ds(j * 16, 16)] = jnp.zeros((16,), jnp.float32)
            return c

        lax.fori_loop(0, CH, zrow, 0)
        for t in range(RPT // CH):
            pltpu.sync_copy(mbufs[0],
                            acc_sh.at[pl.ds(sid * RPT + t * CH, CH)])
        plsc.subcore_barrier()

        def issue(c, b):
            pltpu.async_copy(m_hbm.at[pl.ds(lbase + c * CH, CH)],
                             mbufs[b], lsems[b])
            pltpu.async_copy(src_hbm.at[pl.ds(e0 + lbase + c * CH, CH)],
                             idxbs[b], lsems[b])

        def wait_in(b):
            pltpu.make_async_copy(m_hbm.at[pl.ds(0, CH)], mbufs[b],
                                  lsems[b]).wait()
            pltpu.make_async_copy(src_hbm.at[pl.ds(0, CH)], idxbs[b],
                                  lsems[b]).wait()

        def consume(c, b):
            pltpu.async_copy(mbufs[b], acc_sh.at[idxbs[b]], ssems[b],
                             add=True)

        def wait_out(b):
            pltpu.make_async_copy(mbufs[b], acc_sh.at[idxbs[b]],
                                  ssems[b]).wait()

        _ring_pipeline(nchunk, issue, wait_in, consume, wait_out)
        plsc.subcore_barrier()

        for t in range(RPT // CH):
            row0 = sid * RPT + t * CH
            pltpu.sync_copy(acc_sh.at[pl.ds(row0, CH)],
                            out_hbm.at[pl.ds(cid * NPAD + row0, CH)])

    return body


@functools.lru_cache(maxsize=None)
def _scatter_kernel(h):
    e0, esz = _SPLITS[h]
    return pl.kernel(
        _make_scatter_body(e0, esz // NW),
        out_type=jax.ShapeDtypeStruct((NC * NPAD, D), jnp.float32),
        mesh=_mesh(),
        scratch_types=(
            [pltpu.VMEM((CH,), jnp.int32)] * _RING
            + [pltpu.VMEM((CH, D), jnp.float32)] * _RING
            + [pltpu.VMEM_SHARED((NPAD, D), jnp.float32)]
            + [pltpu.SemaphoreType.DMA] * (2 * _RING)
        ),
    )


def _final_body(*args):
    mp_refs = args[:len(_SPLITS)]
    x_ref, gam_ref, bet_ref, o_ref = args[len(_SPLITS):]
    msg = mp_refs[0][0:N, :] + mp_refs[0][NPAD:NPAD + N, :]
    for mp in mp_refs[1:]:
        msg = msg + mp[0:N, :] + mp[NPAD:NPAD + N, :]
    mean = jnp.mean(msg, axis=0, keepdims=True)
    var = jnp.mean((msg - mean) ** 2, axis=0, keepdims=True)
    norm = (msg - mean) / jnp.sqrt(var + 1e-5) * gam_ref[...] + bet_ref[...]
    o_ref[...] = _softplus(x_ref[...] + norm)


def _final_call(mps, x, gamma, beta):
    return pl.pallas_call(
        _final_body,
        out_shape=jax.ShapeDtypeStruct((N, D), jnp.float32),
    )(*mps, x, gamma, beta)


def kernel(x, edge_attr, edge_source, edge_target, Wf, bf, Ws, bs, gamma, beta):
    src = edge_source.astype(jnp.int32)
    dst = edge_target.astype(jnp.int32)
    b2 = jnp.concatenate([bf, bs]).reshape(1, D2)
    wsrc = jnp.concatenate([Wf[:, :D].T, Ws[:, :D].T], axis=1)
    wdst = jnp.concatenate([Wf[:, D:2 * D].T, Ws[:, D:2 * D].T], axis=1)
    wea = jnp.concatenate([Wf[:, 2 * D:].T, Ws[:, 2 * D:].T], axis=1)

    p, q = _pq_call(x, wsrc, wdst, b2)
    mps = []
    gg = [_gather_kernel(h)(p, q, src, dst) for h in range(len(_SPLITS))]
    for h, (gs, gd) in enumerate(gg):
        m = _edge_call(edge_attr, gs, gd, wea, h)
        mps.append(_scatter_kernel(h)(m, src))
    return _final_call(mps, x, gamma.reshape(1, D), beta.reshape(1, D))


# speedup vs baseline: 1.0989x; 1.0956x over previous
import functools

import jax
import jax.numpy as jnp
from jax import lax
from jax.experimental import pallas as pl
from jax.experimental.pallas import tpu as pltpu
from jax.experimental.pallas import tpu_sc as plsc

N = 10000
E = 320000
D = 128
D2 = 2 * D

NC = 2
NS = 16
NW = NC * NS
_SPLITS = ((0, 168960), (168960, 151040))
CH = 80
_RING = 4
NPAD = 10240
RPT = NPAD // NS


@functools.lru_cache(maxsize=None)
def _mesh():
    return plsc.VectorSubcoreMesh(
        core_axis_name="c", subcore_axis_name="s",
        num_cores=NC, num_subcores=NS)


def _sigmoid(v):
    return 1.0 / (1.0 + jnp.exp(-v))


def _softplus(v):
    return jnp.maximum(v, 0.0) + jnp.log(1.0 + jnp.exp(-jnp.abs(v)))


def _pack2(v):
    lo = lax.bitcast_convert_type(v[:, :D].astype(jnp.bfloat16), jnp.uint16)
    hi = lax.bitcast_convert_type(v[:, D:].astype(jnp.bfloat16), jnp.uint16)
    w = lo.astype(jnp.uint32) | (hi.astype(jnp.uint32) << 16)
    return lax.bitcast_convert_type(w, jnp.int32)


_BN = 2000


def _pq_body(x_ref, ws_ref, wd_ref, b_ref, p_ref, q_ref):
    xb = x_ref[...]
    p = jnp.dot(xb, ws_ref[...],
                preferred_element_type=jnp.float32) + b_ref[...]
    q = jnp.dot(xb, wd_ref[...], preferred_element_type=jnp.float32)
    p_ref[...] = _pack2(p)
    q_ref[...] = _pack2(q)


def _pq_call(x, wsrc, wdst, b2):
    return pl.pallas_call(
        _pq_body,
        grid=(N // _BN,),
        in_specs=[
            pl.BlockSpec((_BN, D), lambda i: (i, 0)),
            pl.BlockSpec((D, D2), lambda i: (0, 0)),
            pl.BlockSpec((D, D2), lambda i: (0, 0)),
            pl.BlockSpec((1, D2), lambda i: (0, 0)),
        ],
        out_specs=[
            pl.BlockSpec((_BN, D), lambda i: (i, 0)),
            pl.BlockSpec((_BN, D), lambda i: (i, 0)),
        ],
        out_shape=[
            jax.ShapeDtypeStruct((N, D), jnp.int32),
            jax.ShapeDtypeStruct((N, D), jnp.int32),
        ],
    )(x, wsrc, wdst, b2)


def _ring_pipeline(nchunk, issue, wait_in, consume, wait_out):
    quads = nchunk // _RING
    tail = nchunk % _RING

    for b in range(_RING - 1):
        issue(b, b)

    def quad(i4, carry):
        for b in range(_RING):
            c = i4 * _RING + b
            tb = (b + _RING - 1) % _RING

            @pl.when(c >= 1)
            def _():
                wait_out(tb)

            @pl.when(c + (_RING - 1) < nchunk)
            def _():
                issue(c + (_RING - 1), tb)

            wait_in(b)
            consume(c, b)
        return carry

    lax.fori_loop(0, quads, quad, 0)

    for c in range(quads * _RING, nchunk):
        b = c % _RING
        wait_out((b + _RING - 1) % _RING)
        if c + (_RING - 1) < nchunk:
            issue(c + (_RING - 1), (b + _RING - 1) % _RING)
        wait_in(b)
        consume(c, b)

    wait_out((nchunk - 1) % _RING)


def _make_gather_body(e0, epw):
    nchunk = epw // CH

    def body(p_hbm, q_hbm, src_hbm, dst_hbm, gs_hbm, gd_hbm, *bufs):
        sidx, didx = bufs[0], bufs[1]
        pbs = bufs[2:2 + _RING]
        qbs = bufs[2 + _RING:2 + 2 * _RING]
        gsems = bufs[2 + 2 * _RING:2 + 3 * _RING]
        wsems = bufs[2 + 3 * _RING:2 + 4 * _RING]

        wid = lax.axis_index("s") * NC + lax.axis_index("c")
        lbase = wid * epw
        pltpu.sync_copy(src_hbm.at[pl.ds(e0 + lbase, epw)], sidx)
        pltpu.sync_copy(dst_hbm.at[pl.ds(e0 + lbase, epw)], didx)

        def issue(c, b):
            isl = pl.ds(c * CH, CH)
            pltpu.async_copy(p_hbm.at[sidx.at[isl]], pbs[b], gsems[b])
            pltpu.async_copy(q_hbm.at[didx.at[isl]], qbs[b], gsems[b])

        def wait_in(b):
            pltpu.make_async_copy(p_hbm.at[sidx.at[pl.ds(0, CH)]], pbs[b],
                                  gsems[b]).wait()
            pltpu.make_async_copy(q_hbm.at[didx.at[pl.ds(0, CH)]], qbs[b],
                                  gsems[b]).wait()

        def consume(c, b):
            osl = pl.ds(lbase + c * CH, CH)
            pltpu.async_copy(pbs[b], gs_hbm.at[osl], wsems[b])
            pltpu.async_copy(qbs[b], gd_hbm.at[osl], wsems[b])

        def wait_out(b):
            pltpu.make_async_copy(pbs[b], gs_hbm.at[pl.ds(0, CH)],
                                  wsems[b]).wait()
            pltpu.make_async_copy(qbs[b], gd_hbm.at[pl.ds(0, CH)],
                                  wsems[b]).wait()

        _ring_pipeline(nchunk, issue, wait_in, consume, wait_out)

    return body


@functools.lru_cache(maxsize=None)
def _gather_kernel(h):
    e0, esz = _SPLITS[h]
    return pl.kernel(
        _make_gather_body(e0, esz // NW),
        out_type=[
            jax.ShapeDtypeStruct((esz, D), jnp.int32),
            jax.ShapeDtypeStruct((esz, D), jnp.int32),
        ],
        mesh=_mesh(),
        scratch_types=(
            [pltpu.VMEM((esz // NW,), jnp.int32)] * 2
            + [pltpu.VMEM((CH, D), jnp.int32)] * (2 * _RING)
            + [pltpu.SemaphoreType.DMA] * (2 * _RING)
        ),
    )


_BE = 2560


def _edge_body(ea_ref, gs_ref, gd_ref, we_ref, m_ref):
    ll = jnp.dot(ea_ref[...], we_ref[...], preferred_element_type=jnp.float32)
    gs = gs_ref[...]
    gd = gd_ref[...]
    lf = (lax.bitcast_convert_type(gs << 16, jnp.float32)
          + lax.bitcast_convert_type(gd << 16, jnp.float32))
    ls = (lax.bitcast_convert_type(gs & jnp.int32(-65536), jnp.float32)
          + lax.bitcast_convert_type(gd & jnp.int32(-65536), jnp.float32))
    f = _sigmoid(ll[:, :D] + lf)
    s = _softplus(ll[:, D:] + ls)
    m_ref[...] = f * s


def _edge_call(ea, gs, gd, wea, h):
    e0, esz = _SPLITS[h]
    hoff = e0 // _BE
    return pl.pallas_call(
        _edge_body,
        grid=(esz // _BE,),
        in_specs=[
            pl.BlockSpec((_BE, D), lambda i: (i + hoff, 0)),
            pl.BlockSpec((_BE, D), lambda i: (i, 0)),
            pl.BlockSpec((_BE, D), lambda i: (i, 0)),
            pl.BlockSpec((D, D2), lambda i: (0, 0)),
        ],
        out_specs=pl.BlockSpec((_BE, D), lambda i: (i, 0)),
        out_shape=jax.ShapeDtypeStruct((esz, D), jnp.float32),
    )(ea, gs, gd, wea)


def _make_scatter_body(e0, epw):
    nchunk = epw // CH

    def body(m_hbm, src_hbm, out_hbm, *bufs):
        idxbs = bufs[:_RING]
        mbufs = bufs[_RING:2 * _RING]
        acc_sh = bufs[2 * _RING]
        lsems = bufs[2 * _RING + 1:3 * _RING + 1]
        ssems = bufs[3 * _RING + 1:4 * _RING + 1]

        cid = lax.axis_index("c")
        sid = lax.axis_index("s")
        wid = sid * NC + cid
        lbase = wid * epw

        def zrow(r, c):
            for j in range(D // 16):
                mbufs[0][r, pl.ds(j * 16, 16)] = jnp.zeros((16,), jnp.float32)
            return c

        lax.fori_loop(0, CH, zrow, 0)
        for t in range(RPT // CH):
            pltpu.sync_copy(mbufs[0],
                            acc_sh.at[pl.ds(sid * RPT + t * CH, CH)])
        plsc.subcore_barrier()

        def issue(c, b):
            pltpu.async_copy(m_hbm.at[pl.ds(lbase + c * CH, CH)],
                             mbufs[b], lsems[b])
            pltpu.async_copy(src_hbm.at[pl.ds(e0 + lbase + c * CH, CH)],
                             idxbs[b], lsems[b])

        def wait_in(b):
            pltpu.make_async_copy(m_hbm.at[pl.ds(0, CH)], mbufs[b],
                                  lsems[b]).wait()
            pltpu.make_async_copy(src_hbm.at[pl.ds(0, CH)], idxbs[b],
                                  lsems[b]).wait()

        def consume(c, b):
            pltpu.async_copy(mbufs[b], acc_sh.at[idxbs[b]], ssems[b],
                             add=True)

        def wait_out(b):
            pltpu.make_async_copy(mbufs[b], acc_sh.at[idxbs[b]],
                                  ssems[b]).wait()

        _ring_pipeline(nchunk, issue, wait_in, consume, wait_out)
        plsc.subcore_barrier()

        for t in range(RPT // CH):
            row0 = sid * RPT + t * CH
            pltpu.sync_copy(acc_sh.at[pl.ds(row0, CH)],
                            out_hbm.at[pl.ds(cid * NPAD + row0, CH)])

    return body


@functools.lru_cache(maxsize=None)
def _scatter_kernel(h):
    e0, esz = _SPLITS[h]
    return pl.kernel(
        _make_scatter_body(e0, esz // NW),
        out_type=jax.ShapeDtypeStruct((NC * NPAD, D), jnp.float32),
        mesh=_mesh(),
        scratch_types=(
            [pltpu.VMEM((CH,), jnp.int32)] * _RING
            + [pltpu.VMEM((CH, D), jnp.float32)] * _RING
            + [pltpu.VMEM_SHARED((NPAD, D), jnp.float32)]
            + [pltpu.SemaphoreType.DMA] * (2 * _RING)
        ),
    )


def _final_body(*args):
    mp_refs = args[:len(_SPLITS)]
    x_ref, gam_ref, bet_ref, o_ref = args[len(_SPLITS):]
    msg = mp_refs[0][0:N, :] + mp_refs[0][NPAD:NPAD + N, :]
    for mp in mp_refs[1:]:
        msg = msg + mp[0:N, :] + mp[NPAD:NPAD + N, :]
    mean = jnp.mean(msg, axis=0, keepdims=True)
    var = jnp.mean((msg - mean) ** 2, axis=0, keepdims=True)
    norm = (msg - mean) / jnp.sqrt(var + 1e-5) * gam_ref[...] + bet_ref[...]
    o_ref[...] = _softplus(x_ref[...] + norm)


def _final_call(mps, x, gamma, beta):
    return pl.pallas_call(
        _final_body,
        out_shape=jax.ShapeDtypeStruct((N, D), jnp.float32),
    )(*mps, x, gamma, beta)


def kernel(x, edge_attr, edge_source, edge_target, Wf, bf, Ws, bs, gamma, beta):
    src = edge_source.astype(jnp.int32)
    dst = edge_target.astype(jnp.int32)
    b2 = jnp.concatenate([bf, bs]).reshape(1, D2)
    wsrc = jnp.concatenate([Wf[:, :D].T, Ws[:, :D].T], axis=1)
    wdst = jnp.concatenate([Wf[:, D:2 * D].T, Ws[:, D:2 * D].T], axis=1)
    wea = jnp.concatenate([Wf[:, 2 * D:].T, Ws[:, 2 * D:].T], axis=1)

    p, q = _pq_call(x, wsrc, wdst, b2)
    mps = []
    gg = [_gather_kernel(h)(p, q, src, dst) for h in range(len(_SPLITS))]
    for h, (gs, gd) in enumerate(gg):
        m = _edge_call(edge_attr, gs, gd, wea, h)
        mps.append(_scatter_kernel(h)(m, src))
    return _final_call(mps, x, gamma.reshape(1, D), beta.reshape(1, D))
